# initial kernel scaffold (unmeasured)
import jax
import jax.numpy as jnp
from jax import lax
from jax.experimental import pallas as pl
from jax.experimental.pallas import tpu as pltpu


def kernel(
    u,
):
    def body(*refs):
        pass

    out_shape = jax.ShapeDtypeStruct(..., jnp.float32)
    return pl.pallas_call(body, out_shape=out_shape)(...)


# baseline (device time: 21924 ns/iter reference)
import jax
import jax.numpy as jnp
from jax import lax
from jax.experimental import pallas as pl
from jax.experimental.pallas import tpu as pltpu


def kernel(u):
    NX, NY, NZ = u.shape
    dtype = u.dtype

    def body(
        u_ref,
        out_ref,
        send_x,
        send_y,
        send_z,
        recv_x,
        recv_y,
        recv_z,
        send_sems,
        recv_sems,
    ):
        cx = lax.axis_index("x")
        cy = lax.axis_index("y")
        cz = lax.axis_index("z")
        nbr_x = (1 - cx, cy, cz)
        nbr_y = (cx, 1 - cy, cz)
        nbr_z = (cx, cy, 1 - cz)

        barrier = pltpu.get_barrier_semaphore()
        for nbr in (nbr_x, nbr_y, nbr_z):
            pl.semaphore_signal(
                barrier, inc=1, device_id=nbr,
                device_id_type=pl.DeviceIdType.MESH,
            )
        pl.semaphore_wait(barrier, 3)

        @pl.when(cx == 0)
        def _():
            send_x[...] = u_ref[NX - 1 : NX, :, :]

        @pl.when(cx == 1)
        def _():
            send_x[...] = u_ref[0:1, :, :]

        @pl.when(cy == 0)
        def _():
            send_y[...] = u_ref[:, NY - 1 : NY, :]

        @pl.when(cy == 1)
        def _():
            send_y[...] = u_ref[:, 0:1, :]

        @pl.when(cz == 0)
        def _():
            send_z[...] = u_ref[:, :, NZ - 1 : NZ]

        @pl.when(cz == 1)
        def _():
            send_z[...] = u_ref[:, :, 0:1]

        rdmas = []
        for a, (sbuf, rbuf, nbr) in enumerate(
            (
                (send_x, recv_x, nbr_x),
                (send_y, recv_y, nbr_y),
                (send_z, recv_z, nbr_z),
            )
        ):
            r = pltpu.make_async_remote_copy(
                src_ref=sbuf,
                dst_ref=rbuf,
                send_sem=send_sems.at[a],
                recv_sem=recv_sems.at[a],
                device_id=nbr,
                device_id_type=pl.DeviceIdType.MESH,
            )
            r.start()
            rdmas.append(r)

        for r in rdmas:
            r.wait()

        u_loc = u_ref[...]

        hx = recv_x[...]
        hy = recv_y[...]
        hz = recv_z[...]
        zero = jnp.zeros((), dtype)
        lo_x = jnp.where(cx == 1, hx, zero)
        hi_x = jnp.where(cx == 0, hx, zero)
        lo_y = jnp.where(cy == 1, hy, zero)
        hi_y = jnp.where(cy == 0, hy, zero)
        lo_z = jnp.where(cz == 1, hz, zero)
        hi_z = jnp.where(cz == 0, hz, zero)

        um_x = jnp.concatenate([lo_x, u_loc[:-1]], axis=0)
        up_x = jnp.concatenate([u_loc[1:], hi_x], axis=0)
        um_y = jnp.concatenate([lo_y, u_loc[:, :-1]], axis=1)
        up_y = jnp.concatenate([u_loc[:, 1:], hi_y], axis=1)
        um_z = jnp.concatenate([lo_z, u_loc[:, :, :-1]], axis=2)
        up_z = jnp.concatenate([u_loc[:, :, 1:], hi_z], axis=2)

        v = um_x + up_x + um_y + up_y + um_z + up_z - 6.0 * u_loc

        gi = lax.broadcasted_iota(jnp.int32, (NX, NY, NZ), 0) + cx * NX
        gj = lax.broadcasted_iota(jnp.int32, (NX, NY, NZ), 1) + cy * NY
        gk = lax.broadcasted_iota(jnp.int32, (NX, NY, NZ), 2) + cz * NZ
        interior = (
            (gi >= 1) & (gi <= 2 * NX - 2)
            & (gj >= 1) & (gj <= 2 * NY - 2)
            & (gk >= 1) & (gk <= 2 * NZ - 2)
        )
        out_ref[...] = jnp.where(interior, v, zero)

    return pl.pallas_call(
        body,
        out_shape=jax.ShapeDtypeStruct((NX, NY, NZ), dtype),
        in_specs=[pl.BlockSpec(memory_space=pltpu.VMEM)],
        out_specs=pl.BlockSpec(memory_space=pltpu.VMEM),
        scratch_shapes=[
            pltpu.VMEM((1, NY, NZ), dtype),
            pltpu.VMEM((NX, 1, NZ), dtype),
            pltpu.VMEM((NX, NY, 1), dtype),
            pltpu.VMEM((1, NY, NZ), dtype),
            pltpu.VMEM((NX, 1, NZ), dtype),
            pltpu.VMEM((NX, NY, 1), dtype),
            pltpu.SemaphoreType.DMA((3,)),
            pltpu.SemaphoreType.DMA((3,)),
        ],
        compiler_params=pltpu.CompilerParams(collective_id=0),
    )(u)


# device time: 20894 ns/iter; 1.0493x vs baseline; 1.0493x over previous
import jax
import jax.numpy as jnp
from jax import lax
from jax.experimental import pallas as pl
from jax.experimental.pallas import tpu as pltpu


def kernel(u):
    NX, NY, NZ = u.shape
    dtype = u.dtype

    def body(
        u_ref,
        out_ref,
        send_x,
        send_y,
        send_z,
        recv_x,
        recv_y,
        recv_z,
        send_sems,
        recv_sems,
    ):
        cx = lax.axis_index("x")
        cy = lax.axis_index("y")
        cz = lax.axis_index("z")
        nbr_x = (1 - cx, cy, cz)
        nbr_y = (cx, 1 - cy, cz)
        nbr_z = (cx, cy, 1 - cz)

        barrier = pltpu.get_barrier_semaphore()
        for nbr in (nbr_x, nbr_y, nbr_z):
            pl.semaphore_signal(
                barrier, inc=1, device_id=nbr,
                device_id_type=pl.DeviceIdType.MESH,
            )
        pl.semaphore_wait(barrier, 3)

        @pl.when(cx == 0)
        def _():
            send_x[...] = u_ref[NX - 1 : NX, :, :]

        @pl.when(cx == 1)
        def _():
            send_x[...] = u_ref[0:1, :, :]

        @pl.when(cy == 0)
        def _():
            send_y[...] = u_ref[:, NY - 1 : NY, :]

        @pl.when(cy == 1)
        def _():
            send_y[...] = u_ref[:, 0:1, :]

        @pl.when(cz == 0)
        def _():
            send_z[...] = u_ref[:, :, NZ - 1 : NZ]

        @pl.when(cz == 1)
        def _():
            send_z[...] = u_ref[:, :, 0:1]

        rdmas = []
        for a, (sbuf, rbuf, nbr) in enumerate(
            (
                (send_x, recv_x, nbr_x),
                (send_y, recv_y, nbr_y),
                (send_z, recv_z, nbr_z),
            )
        ):
            r = pltpu.make_async_remote_copy(
                src_ref=sbuf,
                dst_ref=rbuf,
                send_sem=send_sems.at[a],
                recv_sem=recv_sems.at[a],
                device_id=nbr,
                device_id_type=pl.DeviceIdType.MESH,
            )
            r.start()
            rdmas.append(r)

        u_loc = u_ref[...]
        zx = jnp.zeros((1, NY, NZ), dtype)
        zy = jnp.zeros((NX, 1, NZ), dtype)
        zz = jnp.zeros((NX, NY, 1), dtype)
        um_x = jnp.concatenate([zx, u_loc[:-1]], axis=0)
        up_x = jnp.concatenate([u_loc[1:], zx], axis=0)
        um_y = jnp.concatenate([zy, u_loc[:, :-1]], axis=1)
        up_y = jnp.concatenate([u_loc[:, 1:], zy], axis=1)
        um_z = jnp.concatenate([zz, u_loc[:, :, :-1]], axis=2)
        up_z = jnp.concatenate([u_loc[:, :, 1:], zz], axis=2)
        out_ref[...] = (
            um_x + up_x + um_y + up_y + um_z + up_z - 6.0 * u_loc
        )

        for r in rdmas:
            r.wait()

        @pl.when(cx == 0)
        def _():
            out_ref[NX - 1 : NX, :, :] = out_ref[NX - 1 : NX, :, :] + recv_x[...]

        @pl.when(cx == 1)
        def _():
            out_ref[0:1, :, :] = out_ref[0:1, :, :] + recv_x[...]

        @pl.when(cy == 0)
        def _():
            out_ref[:, NY - 1 : NY, :] = out_ref[:, NY - 1 : NY, :] + recv_y[...]

        @pl.when(cy == 1)
        def _():
            out_ref[:, 0:1, :] = out_ref[:, 0:1, :] + recv_y[...]

        @pl.when(cz == 0)
        def _():
            out_ref[:, :, NZ - 1 : NZ] = out_ref[:, :, NZ - 1 : NZ] + recv_z[...]

        @pl.when(cz == 1)
        def _():
            out_ref[:, :, 0:1] = out_ref[:, :, 0:1] + recv_z[...]

        @pl.when(cx == 0)
        def _():
            out_ref[0:1, :, :] = zx

        @pl.when(cx == 1)
        def _():
            out_ref[NX - 1 : NX, :, :] = zx

        @pl.when(cy == 0)
        def _():
            out_ref[:, 0:1, :] = zy

        @pl.when(cy == 1)
        def _():
            out_ref[:, NY - 1 : NY, :] = zy

        @pl.when(cz == 0)
        def _():
            out_ref[:, :, 0:1] = zz

        @pl.when(cz == 1)
        def _():
            out_ref[:, :, NZ - 1 : NZ] = zz

    return pl.pallas_call(
        body,
        out_shape=jax.ShapeDtypeStruct((NX, NY, NZ), dtype),
        in_specs=[pl.BlockSpec(memory_space=pltpu.VMEM)],
        out_specs=pl.BlockSpec(memory_space=pltpu.VMEM),
        scratch_shapes=[
            pltpu.VMEM((1, NY, NZ), dtype),
            pltpu.VMEM((NX, 1, NZ), dtype),
            pltpu.VMEM((NX, NY, 1), dtype),
            pltpu.VMEM((1, NY, NZ), dtype),
            pltpu.VMEM((NX, 1, NZ), dtype),
            pltpu.VMEM((NX, NY, 1), dtype),
            pltpu.SemaphoreType.DMA((3,)),
            pltpu.SemaphoreType.DMA((3,)),
        ],
        compiler_params=pltpu.CompilerParams(collective_id=0),
    )(u)


# device time: 8698 ns/iter; 2.5206x vs baseline; 2.4022x over previous
import jax
import jax.numpy as jnp
from jax import lax
from jax.experimental import pallas as pl
from jax.experimental.pallas import tpu as pltpu


def kernel(u):
    NX, NY, NZ = u.shape
    dtype = u.dtype

    def body(
        u_ref,
        out_ref,
        send_y,
        send_z,
        recv_x,
        recv_y,
        recv_z,
        send_sems,
        recv_sems,
    ):
        cx = lax.axis_index("x")
        cy = lax.axis_index("y")
        cz = lax.axis_index("z")
        nbr_x = (1 - cx, cy, cz)
        nbr_y = (cx, 1 - cy, cz)
        nbr_z = (cx, cy, 1 - cz)

        barrier = pltpu.get_barrier_semaphore()
        for nbr in (nbr_x, nbr_y, nbr_z):
            pl.semaphore_signal(
                barrier, inc=1, device_id=nbr,
                device_id_type=pl.DeviceIdType.MESH,
            )

        u_loc = u_ref[...]
        ix = jnp.where(cx == 0, NX - 1, 0)

        @pl.when(cy == 0)
        def _():
            send_y[...] = u_loc[:, NY - 1, :]

        @pl.when(cy == 1)
        def _():
            send_y[...] = u_loc[:, 0, :]

        @pl.when(cz == 0)
        def _():
            send_z[...] = u_loc[:, :, NZ - 1]

        @pl.when(cz == 1)
        def _():
            send_z[...] = u_loc[:, :, 0]

        pl.semaphore_wait(barrier, 3)

        rdmas = []
        for a, (sbuf, rbuf, nbr) in enumerate(
            (
                (u_ref.at[pl.ds(ix, 1)], recv_x, nbr_x),
                (send_y, recv_y, nbr_y),
                (send_z, recv_z, nbr_z),
            )
        ):
            r = pltpu.make_async_remote_copy(
                src_ref=sbuf,
                dst_ref=rbuf,
                send_sem=send_sems.at[a],
                recv_sem=recv_sems.at[a],
                device_id=nbr,
                device_id_type=pl.DeviceIdType.MESH,
            )
            r.start()
            rdmas.append(r)

        zx = jnp.zeros((1, NY, NZ), dtype)
        zy = jnp.zeros((NX, 1, NZ), dtype)
        zz = jnp.zeros((NX, NY, 1), dtype)
        um_x = jnp.concatenate([zx, u_loc[:-1]], axis=0)
        up_x = jnp.concatenate([u_loc[1:], zx], axis=0)
        um_y = jnp.concatenate([zy, u_loc[:, :-1]], axis=1)
        up_y = jnp.concatenate([u_loc[:, 1:], zy], axis=1)
        um_z = jnp.concatenate([zz, u_loc[:, :, :-1]], axis=2)
        up_z = jnp.concatenate([u_loc[:, :, 1:], zz], axis=2)
        out_ref[...] = (
            um_x + up_x + um_y + up_y + um_z + up_z - 6.0 * u_loc
        )

        for r in rdmas:
            r.wait_recv()

        @pl.when(cx == 0)
        def _():
            out_ref[NX - 1 : NX, :, :] = out_ref[NX - 1 : NX, :, :] + recv_x[...]

        @pl.when(cx == 1)
        def _():
            out_ref[0:1, :, :] = out_ref[0:1, :, :] + recv_x[...]

        @pl.when(cy == 0)
        def _():
            out_ref[:, NY - 1 : NY, :] = (
                out_ref[:, NY - 1 : NY, :] + recv_y[...][:, None, :]
            )

        @pl.when(cy == 1)
        def _():
            out_ref[:, 0:1, :] = out_ref[:, 0:1, :] + recv_y[...][:, None, :]

        @pl.when(cz == 0)
        def _():
            out_ref[:, :, NZ - 1 : NZ] = (
                out_ref[:, :, NZ - 1 : NZ] + recv_z[...][:, :, None]
            )

        @pl.when(cz == 1)
        def _():
            out_ref[:, :, 0:1] = out_ref[:, :, 0:1] + recv_z[...][:, :, None]

        @pl.when(cx == 0)
        def _():
            out_ref[0:1, :, :] = zx

        @pl.when(cx == 1)
        def _():
            out_ref[NX - 1 : NX, :, :] = zx

        @pl.when(cy == 0)
        def _():
            out_ref[:, 0:1, :] = zy

        @pl.when(cy == 1)
        def _():
            out_ref[:, NY - 1 : NY, :] = zy

        @pl.when(cz == 0)
        def _():
            out_ref[:, :, 0:1] = zz

        @pl.when(cz == 1)
        def _():
            out_ref[:, :, NZ - 1 : NZ] = zz

        for r in rdmas:
            r.wait_send()

    return pl.pallas_call(
        body,
        out_shape=jax.ShapeDtypeStruct((NX, NY, NZ), dtype),
        in_specs=[pl.BlockSpec(memory_space=pltpu.VMEM)],
        out_specs=pl.BlockSpec(memory_space=pltpu.VMEM),
        scratch_shapes=[
            pltpu.VMEM((NX, NZ), dtype),
            pltpu.VMEM((NX, NY), dtype),
            pltpu.VMEM((1, NY, NZ), dtype),
            pltpu.VMEM((NX, NZ), dtype),
            pltpu.VMEM((NX, NY), dtype),
            pltpu.SemaphoreType.DMA((3,)),
            pltpu.SemaphoreType.DMA((3,)),
        ],
        compiler_params=pltpu.CompilerParams(collective_id=0),
    )(u)
